# baseline (speedup 1.0000x reference)
import functools

import jax
import jax.numpy as jnp
from jax import lax
from jax.experimental import pallas as pl
from jax.experimental.pallas import tpu as pltpu
from jax.experimental.pallas import tpu_sc as plsc

_K = 8
_STAT_W = 0.001
_NB = 64
_NW = 32
_LANES = 16
_BT = 4096
_CHUNKS = 2


def _tc_body(tpw, x_ref, w_ref, b_ref, o_ref):
    lg = lax.dot_general(
        w_ref[...], x_ref[...], (((1,), (1,)), ((), ())),
        preferred_element_type=jnp.float32,
    ) + b_ref[...]
    for q in range(_BT // tpw):
        o_ref[q] = lg[:, q * tpw:(q + 1) * tpw]


def _sc_body(tpw, lg_hbm, tp_hbm, ti_hbm, acc_hbm, lg_v, tp_v, ti_v, acc_v):
    cid = lax.axis_index("c")
    sid = lax.axis_index("s")
    wid = sid * 2 + cid

    pltpu.sync_copy(lg_hbm.at[wid], lg_v)

    lane = lax.iota(jnp.int32, _LANES)
    zero16 = jnp.zeros((_LANES,), jnp.float32)

    def zero_acc(r, carry):
        plsc.store_scatter(acc_v, [r * _LANES + lane], zero16)
        return carry

    lax.fori_loop(0, _NB, zero_acc, 0)

    neg = jnp.full((_LANES,), jnp.int32(-(2**31)), jnp.int32)
    m31 = jnp.int32(0x7FFFFFFF)
    mhi = jnp.int32(~63)

    def group(g, carry):
        row = g * _LANES + lane
        t = [neg] * _K
        for c in range(_NB):
            v = plsc.load_gather(lg_v, [jnp.full((_LANES,), c, jnp.int32), row])
            iv = plsc.bitcast(v, jnp.int32)
            key = iv ^ (lax.shift_right_arithmetic(iv, 31) & m31)
            key = (key & mhi) | jnp.int32(_NB - 1 - c)
            for j in range(_K):
                hi = jnp.maximum(t[j], key)
                key = jnp.minimum(t[j], key)
                t[j] = hi
        cols, vals = [], []
        for j in range(_K):
            cols.append(jnp.int32(_NB - 1) - (t[j] & jnp.int32(63)))
            vb = (t[j] & mhi) | jnp.int32(32)
            ib = vb ^ (lax.shift_right_arithmetic(vb, 31) & m31)
            vals.append(plsc.bitcast(ib, jnp.float32))
        es = [jnp.exp(v - vals[0]) for v in vals]
        z = es[0]
        for j in range(1, _K):
            z = z + es[j]
        rz = 1.0 / z
        obase = row * _K
        for j in range(_K):
            p = es[j] * rz
            plsc.store_scatter(tp_v, [obase + j], p)
            plsc.store_scatter(ti_v, [obase + j], cols[j])
            plsc.addupdate_scatter(acc_v, [cols[j] * _LANES + lane], p)
        return carry

    lax.fori_loop(0, tpw // _LANES, group, 0)

    pltpu.sync_copy(tp_v, tp_hbm.at[pl.ds(wid * tpw * _K, tpw * _K)])
    pltpu.sync_copy(ti_v, ti_hbm.at[pl.ds(wid * tpw * _K, tpw * _K)])
    pltpu.sync_copy(acc_v, acc_hbm.at[wid])


def _run_chunk(x, W, b2, n_banks, d_model):
    n_tok = x.shape[0]
    tpw = n_tok // _NW

    logits3 = pl.pallas_call(
        functools.partial(_tc_body, tpw),
        grid=(n_tok // _BT,),
        in_specs=[
            pl.BlockSpec((_BT, d_model), lambda i: (i, 0)),
            pl.BlockSpec((n_banks, d_model), lambda i: (0, 0)),
            pl.BlockSpec((n_banks, 1), lambda i: (0, 0)),
        ],
        out_specs=pl.BlockSpec((_BT // tpw, n_banks, tpw), lambda i: (i, 0, 0)),
        out_shape=jax.ShapeDtypeStruct((_NW, n_banks, tpw), jnp.float32),
    )(x, W, b2)

    sc_call = functools.partial(
        pl.kernel,
        out_type=[
            jax.ShapeDtypeStruct((n_tok * _K,), jnp.float32),
            jax.ShapeDtypeStruct((n_tok * _K,), jnp.int32),
            jax.ShapeDtypeStruct((_NW, _NB * _LANES), jnp.float32),
        ],
        mesh=plsc.VectorSubcoreMesh(core_axis_name="c", subcore_axis_name="s"),
        compiler_params=pltpu.CompilerParams(needs_layout_passes=False),
        scratch_types=[
            pltpu.VMEM((n_banks, tpw), jnp.float32),
            pltpu.VMEM((tpw * _K,), jnp.float32),
            pltpu.VMEM((tpw * _K,), jnp.int32),
            pltpu.VMEM((_NB * _LANES,), jnp.float32),
        ],
    )(functools.partial(_sc_body, tpw))

    tp, ti, parts = sc_call(logits3)
    return tp.reshape(n_tok, _K), ti.reshape(n_tok, _K), parts


def kernel(tensor, W, b, bank_statistics, top_k):
    n_tokens, d_model = tensor.shape
    n_banks = W.shape[0]
    b2 = b.reshape(n_banks, 1)
    csz = n_tokens // _CHUNKS

    tps, tis, parts_list = [], [], []
    for k in range(_CHUNKS):
        tp_c, ti_c, parts_c = _run_chunk(
            tensor[k * csz:(k + 1) * csz], W, b2, n_banks, d_model)
        tps.append(tp_c)
        tis.append(ti_c)
        parts_list.append(parts_c)

    tp = jnp.concatenate(tps, axis=0) if _CHUNKS > 1 else tps[0]
    ti = jnp.concatenate(tis, axis=0) if _CHUNKS > 1 else tis[0]
    ti = ti + (jnp.asarray(top_k, jnp.int32) - _K)
    acc = sum(p.reshape(_NW, n_banks, _LANES).sum(axis=(0, 2))
              for p in parts_list)
    stats = bank_statistics * (1.0 - _STAT_W) + acc * (_STAT_W / n_tokens)
    return tp, ti, stats


# speedup vs baseline: 1.2640x; 1.2640x over previous
import functools

import jax
import jax.numpy as jnp
from jax import lax
from jax.experimental import pallas as pl
from jax.experimental.pallas import tpu as pltpu
from jax.experimental.pallas import tpu_sc as plsc

_K = 8
_STAT_W = 0.001
_NB = 64
_NW = 32
_LANES = 16
_BT = 4096
_CHUNKS = 1


def _tc_body(tpw, x_ref, w_ref, b_ref, o_ref):
    lg = lax.dot_general(
        w_ref[...], x_ref[...], (((1,), (1,)), ((), ())),
        preferred_element_type=jnp.float32,
    ) + b_ref[...]
    for q in range(_BT // tpw):
        o_ref[q] = lg[:, q * tpw:(q + 1) * tpw]


def _sc_body(tpw, lg_hbm, tp_hbm, ti_hbm, acc_hbm, lg_v, tp_v, ti_v, acc_v):
    cid = lax.axis_index("c")
    sid = lax.axis_index("s")
    wid = sid * 2 + cid

    pltpu.sync_copy(lg_hbm.at[wid], lg_v)

    lane = lax.iota(jnp.int32, _LANES)
    zero16 = jnp.zeros((_LANES,), jnp.float32)

    def zero_acc(r, carry):
        plsc.store_scatter(acc_v, [r * _LANES + lane], zero16)
        return carry

    lax.fori_loop(0, _NB, zero_acc, 0)

    neg = jnp.full((_LANES,), jnp.int32(-(2**31)), jnp.int32)
    m31 = jnp.int32(0x7FFFFFFF)
    mhi = jnp.int32(~63)

    def group(g, carry):
        row = g * _LANES + lane
        t = [neg] * _K
        for c in range(_NB):
            v = plsc.load_gather(lg_v, [jnp.full((_LANES,), c, jnp.int32), row])
            iv = plsc.bitcast(v, jnp.int32)
            key = iv ^ (lax.shift_right_arithmetic(iv, 31) & m31)
            key = (key & mhi) | jnp.int32(_NB - 1 - c)
            for j in range(_K):
                hi = jnp.maximum(t[j], key)
                key = jnp.minimum(t[j], key)
                t[j] = hi
        cols, vals = [], []
        for j in range(_K):
            cols.append(jnp.int32(_NB - 1) - (t[j] & jnp.int32(63)))
            vb = (t[j] & mhi) | jnp.int32(32)
            ib = vb ^ (lax.shift_right_arithmetic(vb, 31) & m31)
            vals.append(plsc.bitcast(ib, jnp.float32))
        es = [jnp.exp(v - vals[0]) for v in vals]
        z = es[0]
        for j in range(1, _K):
            z = z + es[j]
        rz = 1.0 / z
        for j in range(_K):
            p = es[j] * rz
            jcol = jnp.full((_LANES,), j, jnp.int32)
            plsc.store_scatter(tp_v, [row, jcol], p)
            plsc.store_scatter(ti_v, [row, jcol], cols[j])
            plsc.addupdate_scatter(acc_v, [cols[j] * _LANES + lane], p)
        return carry

    lax.fori_loop(0, tpw // _LANES, group, 0)

    pltpu.sync_copy(tp_v, tp_hbm.at[pl.ds(wid * tpw, tpw), :])
    pltpu.sync_copy(ti_v, ti_hbm.at[pl.ds(wid * tpw, tpw), :])
    pltpu.sync_copy(acc_v, acc_hbm.at[wid])


def _run_chunk(x, W, b2, n_banks, d_model):
    n_tok = x.shape[0]
    tpw = n_tok // _NW

    logits3 = pl.pallas_call(
        functools.partial(_tc_body, tpw),
        grid=(n_tok // _BT,),
        in_specs=[
            pl.BlockSpec((_BT, d_model), lambda i: (i, 0)),
            pl.BlockSpec((n_banks, d_model), lambda i: (0, 0)),
            pl.BlockSpec((n_banks, 1), lambda i: (0, 0)),
        ],
        out_specs=pl.BlockSpec((_BT // tpw, n_banks, tpw), lambda i: (i, 0, 0)),
        out_shape=jax.ShapeDtypeStruct((_NW, n_banks, tpw), jnp.float32),
    )(x, W, b2)

    sc_call = functools.partial(
        pl.kernel,
        out_type=[
            jax.ShapeDtypeStruct((n_tok, _K), jnp.float32),
            jax.ShapeDtypeStruct((n_tok, _K), jnp.int32),
            jax.ShapeDtypeStruct((_NW, _NB * _LANES), jnp.float32),
        ],
        mesh=plsc.VectorSubcoreMesh(core_axis_name="c", subcore_axis_name="s"),
        compiler_params=pltpu.CompilerParams(
            needs_layout_passes=False, use_tc_tiling_on_sc=False),
        scratch_types=[
            pltpu.VMEM((n_banks, tpw), jnp.float32),
            pltpu.VMEM((tpw, _K), jnp.float32),
            pltpu.VMEM((tpw, _K), jnp.int32),
            pltpu.VMEM((_NB * _LANES,), jnp.float32),
        ],
    )(functools.partial(_sc_body, tpw))

    tp, ti, parts = sc_call(logits3)
    return tp, ti, parts


def kernel(tensor, W, b, bank_statistics, top_k):
    n_tokens, d_model = tensor.shape
    n_banks = W.shape[0]
    b2 = b.reshape(n_banks, 1)
    csz = n_tokens // _CHUNKS

    tps, tis, parts_list = [], [], []
    for k in range(_CHUNKS):
        tp_c, ti_c, parts_c = _run_chunk(
            tensor[k * csz:(k + 1) * csz], W, b2, n_banks, d_model)
        tps.append(tp_c)
        tis.append(ti_c)
        parts_list.append(parts_c)

    tp = jnp.concatenate(tps, axis=0) if _CHUNKS > 1 else tps[0]
    ti = jnp.concatenate(tis, axis=0) if _CHUNKS > 1 else tis[0]
    ti = ti + (jnp.asarray(top_k, jnp.int32) - _K)
    acc = sum(p.reshape(_NW, n_banks, _LANES).sum(axis=(0, 2))
              for p in parts_list)
    stats = bank_statistics * (1.0 - _STAT_W) + acc * (_STAT_W / n_tokens)
    return tp, ti, stats


# speedup vs baseline: 1.5576x; 1.2323x over previous
import functools

import jax
import jax.numpy as jnp
from jax import lax
from jax.experimental import pallas as pl
from jax.experimental.pallas import tpu as pltpu
from jax.experimental.pallas import tpu_sc as plsc

_K = 8
_STAT_W = 0.001
_NB = 64
_NW = 32
_LANES = 16
_BT = 4096
_CHUNKS = 1


def _tc_body(tpw, x_ref, w_ref, b_ref, o_ref):
    lg = lax.dot_general(
        w_ref[...], x_ref[...], (((1,), (1,)), ((), ())),
        preferred_element_type=jnp.float32,
    ) + b_ref[...]
    iv = lax.bitcast_convert_type(lg, jnp.int32)
    key = iv ^ (lax.shift_right_arithmetic(iv, 31) & jnp.int32(0x7FFFFFFF))
    bank = lax.broadcasted_iota(jnp.int32, lg.shape, 0)
    key = (key & jnp.int32(~63)) | (jnp.int32(_NB - 1) - bank)
    for q in range(_BT // tpw):
        o_ref[q] = key[:, q * tpw:(q + 1) * tpw]


def _sc_body(tpw, lg_hbm, tp_hbm, ti_hbm, acc_hbm, lg_v, tp_v, ti_v, acc_v):
    cid = lax.axis_index("c")
    sid = lax.axis_index("s")
    wid = sid * 2 + cid

    pltpu.sync_copy(lg_hbm.at[wid], lg_v)

    lane = lax.iota(jnp.int32, _LANES)
    zero16 = jnp.zeros((_LANES,), jnp.float32)

    def zero_acc(r, carry):
        plsc.store_scatter(acc_v, [r * _LANES + lane], zero16)
        return carry

    lax.fori_loop(0, _NB, zero_acc, 0)

    neg = jnp.full((_LANES,), jnp.int32(-(2**31)), jnp.int32)
    m31 = jnp.int32(0x7FFFFFFF)
    mhi = jnp.int32(~63)

    def group(g, carry):
        row = g * _LANES + lane
        t = [neg] * _K
        for c in range(_NB):
            key = plsc.load_gather(
                lg_v, [jnp.full((_LANES,), c, jnp.int32), row])
            for j in range(_K):
                hi = jnp.maximum(t[j], key)
                key = jnp.minimum(t[j], key)
                t[j] = hi
        cols, vals = [], []
        for j in range(_K):
            cols.append(jnp.int32(_NB - 1) - (t[j] & jnp.int32(63)))
            vb = (t[j] & mhi) | jnp.int32(32)
            ib = vb ^ (lax.shift_right_arithmetic(vb, 31) & m31)
            vals.append(plsc.bitcast(ib, jnp.float32))
        es = [jnp.exp(v - vals[0]) for v in vals]
        z = es[0]
        for j in range(1, _K):
            z = z + es[j]
        rz = 1.0 / z
        obase = row * _K
        for j in range(_K):
            p = es[j] * rz
            plsc.store_scatter(tp_v, [obase + j], p)
            plsc.store_scatter(ti_v, [obase + j], cols[j])
            plsc.addupdate_scatter(acc_v, [cols[j] * _LANES + lane], p)
        return carry

    lax.fori_loop(0, tpw // _LANES, group, 0)

    pltpu.sync_copy(tp_v, tp_hbm.at[pl.ds(wid * tpw * _K, tpw * _K)])
    pltpu.sync_copy(ti_v, ti_hbm.at[pl.ds(wid * tpw * _K, tpw * _K)])
    pltpu.sync_copy(acc_v, acc_hbm.at[wid])


def _run_chunk(x, W, b2, n_banks, d_model):
    n_tok = x.shape[0]
    tpw = n_tok // _NW

    logits3 = pl.pallas_call(
        functools.partial(_tc_body, tpw),
        grid=(n_tok // _BT,),
        in_specs=[
            pl.BlockSpec((_BT, d_model), lambda i: (i, 0)),
            pl.BlockSpec((n_banks, d_model), lambda i: (0, 0)),
            pl.BlockSpec((n_banks, 1), lambda i: (0, 0)),
        ],
        out_specs=pl.BlockSpec((_BT // tpw, n_banks, tpw), lambda i: (i, 0, 0)),
        out_shape=jax.ShapeDtypeStruct((_NW, n_banks, tpw), jnp.int32),
    )(x, W, b2)

    sc_call = functools.partial(
        pl.kernel,
        out_type=[
            jax.ShapeDtypeStruct((n_tok * _K,), jnp.float32),
            jax.ShapeDtypeStruct((n_tok * _K,), jnp.int32),
            jax.ShapeDtypeStruct((_NW, _NB * _LANES), jnp.float32),
        ],
        mesh=plsc.VectorSubcoreMesh(core_axis_name="c", subcore_axis_name="s"),
        compiler_params=pltpu.CompilerParams(needs_layout_passes=False),
        scratch_types=[
            pltpu.VMEM((n_banks, tpw), jnp.int32),
            pltpu.VMEM((tpw * _K,), jnp.float32),
            pltpu.VMEM((tpw * _K,), jnp.int32),
            pltpu.VMEM((_NB * _LANES,), jnp.float32),
        ],
    )(functools.partial(_sc_body, tpw))

    tp, ti, parts = sc_call(logits3)
    return tp.reshape(n_tok, _K), ti.reshape(n_tok, _K), parts


def kernel(tensor, W, b, bank_statistics, top_k):
    n_tokens, d_model = tensor.shape
    n_banks = W.shape[0]
    b2 = b.reshape(n_banks, 1)
    csz = n_tokens // _CHUNKS

    tps, tis, parts_list = [], [], []
    for k in range(_CHUNKS):
        tp_c, ti_c, parts_c = _run_chunk(
            tensor[k * csz:(k + 1) * csz], W, b2, n_banks, d_model)
        tps.append(tp_c)
        tis.append(ti_c)
        parts_list.append(parts_c)

    tp = jnp.concatenate(tps, axis=0) if _CHUNKS > 1 else tps[0]
    ti = jnp.concatenate(tis, axis=0) if _CHUNKS > 1 else tis[0]
    ti = ti + (jnp.asarray(top_k, jnp.int32) - _K)
    acc = sum(p.reshape(_NW, n_banks, _LANES).sum(axis=(0, 2))
              for p in parts_list)
    stats = bank_statistics * (1.0 - _STAT_W) + acc * (_STAT_W / n_tokens)
    return tp, ti, stats


# speedup vs baseline: 1.5583x; 1.0004x over previous
import functools

import jax
import jax.numpy as jnp
from jax import lax
from jax.experimental import pallas as pl
from jax.experimental.pallas import tpu as pltpu
from jax.experimental.pallas import tpu_sc as plsc

_K = 8
_STAT_W = 0.001
_NB = 64
_NW = 32
_LANES = 16
_BT = 4096
_CHUNKS = 1


def _tc_body(tpw, x_ref, w_ref, b_ref, o_ref):
    lg = lax.dot_general(
        w_ref[...], x_ref[...], (((1,), (1,)), ((), ())),
        preferred_element_type=jnp.float32,
    ) + b_ref[...]
    iv = lax.bitcast_convert_type(lg, jnp.int32)
    key = iv ^ (lax.shift_right_arithmetic(iv, 31) & jnp.int32(0x7FFFFFFF))
    bank = lax.broadcasted_iota(jnp.int32, lg.shape, 0)
    key = ((key + jnp.int32(32)) & jnp.int32(~63)) | (jnp.int32(_NB - 1) - bank)
    for q in range(_BT // tpw):
        o_ref[q] = key[:, q * tpw:(q + 1) * tpw]


def _sc_body(tpw, lg_hbm, tp_hbm, ti_hbm, acc_hbm, lg_v, tp_v, ti_v, acc_v):
    cid = lax.axis_index("c")
    sid = lax.axis_index("s")
    wid = sid * 2 + cid

    pltpu.sync_copy(lg_hbm.at[wid], lg_v)

    lane = lax.iota(jnp.int32, _LANES)
    zero16 = jnp.zeros((_LANES,), jnp.float32)

    def zero_acc(r, carry):
        plsc.store_scatter(acc_v, [r * _LANES + lane], zero16)
        return carry

    lax.fori_loop(0, _NB, zero_acc, 0)

    neg = jnp.full((_LANES,), jnp.int32(-(2**31)), jnp.int32)
    m31 = jnp.int32(0x7FFFFFFF)
    mhi = jnp.int32(~63)

    def group(g, carry):
        row = g * _LANES + lane
        t = [neg] * _K
        for c in range(_NB):
            key = plsc.load_gather(
                lg_v, [jnp.full((_LANES,), c, jnp.int32), row])
            for j in range(_K):
                hi = jnp.maximum(t[j], key)
                key = jnp.minimum(t[j], key)
                t[j] = hi
        cols, vals = [], []
        for j in range(_K):
            cols.append(jnp.int32(_NB - 1) - (t[j] & jnp.int32(63)))
            vb = t[j] & mhi
            ib = vb ^ (lax.shift_right_arithmetic(vb, 31) & m31)
            vals.append(plsc.bitcast(ib, jnp.float32))
        es = [jnp.exp(v - vals[0]) for v in vals]
        z = es[0]
        for j in range(1, _K):
            z = z + es[j]
        rz = 1.0 / z
        obase = row * _K
        for j in range(_K):
            p = es[j] * rz
            plsc.store_scatter(tp_v, [obase + j], p)
            plsc.store_scatter(ti_v, [obase + j], cols[j])
            plsc.addupdate_scatter(acc_v, [cols[j] * _LANES + lane], p)
        return carry

    lax.fori_loop(0, tpw // _LANES, group, 0)

    pltpu.sync_copy(tp_v, tp_hbm.at[pl.ds(wid * tpw * _K, tpw * _K)])
    pltpu.sync_copy(ti_v, ti_hbm.at[pl.ds(wid * tpw * _K, tpw * _K)])
    pltpu.sync_copy(acc_v, acc_hbm.at[wid])


def _run_chunk(x, W, b2, n_banks, d_model):
    n_tok = x.shape[0]
    tpw = n_tok // _NW

    logits3 = pl.pallas_call(
        functools.partial(_tc_body, tpw),
        grid=(n_tok // _BT,),
        in_specs=[
            pl.BlockSpec((_BT, d_model), lambda i: (i, 0)),
            pl.BlockSpec((n_banks, d_model), lambda i: (0, 0)),
            pl.BlockSpec((n_banks, 1), lambda i: (0, 0)),
        ],
        out_specs=pl.BlockSpec((_BT // tpw, n_banks, tpw), lambda i: (i, 0, 0)),
        out_shape=jax.ShapeDtypeStruct((_NW, n_banks, tpw), jnp.int32),
    )(x, W, b2)

    sc_call = functools.partial(
        pl.kernel,
        out_type=[
            jax.ShapeDtypeStruct((n_tok * _K,), jnp.float32),
            jax.ShapeDtypeStruct((n_tok * _K,), jnp.int32),
            jax.ShapeDtypeStruct((_NW, _NB * _LANES), jnp.float32),
        ],
        mesh=plsc.VectorSubcoreMesh(core_axis_name="c", subcore_axis_name="s"),
        compiler_params=pltpu.CompilerParams(needs_layout_passes=False),
        scratch_types=[
            pltpu.VMEM((n_banks, tpw), jnp.int32),
            pltpu.VMEM((tpw * _K,), jnp.float32),
            pltpu.VMEM((tpw * _K,), jnp.int32),
            pltpu.VMEM((_NB * _LANES,), jnp.float32),
        ],
    )(functools.partial(_sc_body, tpw))

    tp, ti, parts = sc_call(logits3)
    return tp.reshape(n_tok, _K), ti.reshape(n_tok, _K), parts


def kernel(tensor, W, b, bank_statistics, top_k):
    n_tokens, d_model = tensor.shape
    n_banks = W.shape[0]
    b2 = b.reshape(n_banks, 1)
    csz = n_tokens // _CHUNKS

    tps, tis, parts_list = [], [], []
    for k in range(_CHUNKS):
        tp_c, ti_c, parts_c = _run_chunk(
            tensor[k * csz:(k + 1) * csz], W, b2, n_banks, d_model)
        tps.append(tp_c)
        tis.append(ti_c)
        parts_list.append(parts_c)

    tp = jnp.concatenate(tps, axis=0) if _CHUNKS > 1 else tps[0]
    ti = jnp.concatenate(tis, axis=0) if _CHUNKS > 1 else tis[0]
    ti = ti + (jnp.asarray(top_k, jnp.int32) - _K)
    acc = sum(p.reshape(_NW, n_banks, _LANES).sum(axis=(0, 2))
              for p in parts_list)
    stats = bank_statistics * (1.0 - _STAT_W) + acc * (_STAT_W / n_tokens)
    return tp, ti, stats
